# Initial kernel scaffold; baseline (speedup 1.0000x reference)
import jax
import jax.numpy as jnp
from jax.experimental import pallas as pl


def kernel(node_attr, edge_index, edge_attr, edge_sh, W_fc1, b_fc1, W_fc2, b_fc2, W1, W2, W3, W4):
    raise NotImplementedError("write your pallas kernel here")


# speedup vs baseline: 2.1902x; 2.1902x over previous
import functools
import math

import jax
import jax.numpy as jnp
from jax import lax
from jax.experimental import pallas as pl
from jax.experimental.pallas import tpu as pltpu
from jax.experimental.pallas import tpu_sc as plsc

N_NODES = 50000
N_EDGES = 800000
MUL_S = 32
MUL_V = 16
D_IN = 80
D_PAD = 128
D_EDGE = 16
N_FEAT = 64
HALF = 40

_NW = 32
_KG = 128
_EH = (384000, 416000)


def _make_gather(ne):
    epw = ne // _NW
    nfg = epw // _KG
    tg = epw - nfg * _KG
    assert nfg % 2 == 1 and tg % 8 == 0
    mesh = plsc.VectorSubcoreMesh(core_axis_name="c", subcore_axis_name="s")

    @functools.partial(
        pl.kernel,
        out_type=jax.ShapeDtypeStruct((ne, D_PAD), jnp.float32),
        mesh=mesh,
        scratch_types=[
            pltpu.VMEM((_KG,), jnp.int32),
            pltpu.VMEM((_KG, D_PAD), jnp.float32),
            pltpu.VMEM((_KG,), jnp.int32),
            pltpu.VMEM((_KG, D_PAD), jnp.float32),
            pltpu.VMEM((tg,), jnp.int32),
            pltpu.VMEM((tg, D_PAD), jnp.float32),
            pltpu.SemaphoreType.DMA,
            pltpu.SemaphoreType.DMA,
        ],
    )
    def gather_k(nodes_hbm, idx_hbm, x_hbm, idxa_v, rowsa_v, idxb_v, rowsb_v,
                 idxt_v, rowst_v, sema, semb):
        cid = lax.axis_index("c")
        sid = lax.axis_index("s")
        wid = sid * 2 + cid
        base0 = wid * epw
        bufs = ((idxa_v, rowsa_v, sema), (idxb_v, rowsb_v, semb))

        def start(j, idx_v, rows_v, sem):
            base = pl.multiple_of(base0 + j * _KG, 8)
            pltpu.sync_copy(idx_hbm.at[pl.ds(base, _KG)], idx_v)
            pltpu.async_copy(nodes_hbm.at[idx_v], rows_v, sem)

        def finish(j, idx_v, rows_v, sem):
            base = pl.multiple_of(base0 + j * _KG, 8)
            pltpu.make_async_copy(nodes_hbm.at[idx_v], rows_v, sem).wait()
            pltpu.sync_copy(rows_v, x_hbm.at[pl.ds(base, _KG)])

        start(0, *bufs[0])

        def body(k, carry):
            for b in range(2):
                start(2 * k + 1 + b, *bufs[1 - b])
                finish(2 * k + b, *bufs[b])
            return carry

        lax.fori_loop(0, (nfg - 1) // 2, body, 0)
        finish(nfg - 1, *bufs[0])
        base = pl.multiple_of(base0 + nfg * _KG, 8)
        pltpu.sync_copy(idx_hbm.at[pl.ds(base, tg)], idxt_v)
        pltpu.async_copy(nodes_hbm.at[idxt_v], rowst_v, sema).wait()
        pltpu.sync_copy(rowst_v, x_hbm.at[pl.ds(base, tg)])

    return gather_k


_BE = 2000


_dot = functools.partial(jnp.dot, preferred_element_type=jnp.float32)


def _tc_body(x_ref, ea_ref, sh_ref, wfc1, bfc1, wfc2, bfc2, r2g, mfused, q48,
             out_ref):
    x = x_ref[...]
    sh = sh_ref[...]
    s0 = sh[:, 0:1]
    s1c = [sh[:, 1 + c:2 + c] for c in range(3)]

    inv_s = 1.0 / math.sqrt(MUL_S)
    inv_d = 1.0 / math.sqrt(MUL_V * 3)
    inv_v = 1.0 / math.sqrt(MUL_V)
    inv_2 = 1.0 / math.sqrt(2.0)

    h = jnp.maximum(_dot(ea_ref[...], wfc1[...]) + bfc1[...], 0.0)
    w = _dot(h, wfc2[...]) + bfc2[...]

    cm = (lax.broadcasted_iota(jnp.int32, (1, D_PAD), 1) - MUL_S) % 3
    m1 = jnp.where(cm == 0, s1c[0], jnp.where(cm == 1, s1c[1], s1c[2]))
    p2 = _dot(x * m1, r2g[...]) * (w[:, 1:2] * (inv_d * inv_2))

    big = _dot(x, mfused[...])
    p1 = big[:, 0:MUL_S] * (s0 * (w[:, 0:1] * (inv_s * inv_2)))
    b4 = big[:, MUL_S:MUL_S + 48]
    a3 = big[:, MUL_S + 48:]
    out_ref[:, 0:MUL_S] = p1 + p2

    c2 = w[:, 2:3] * (inv_s * inv_2)
    c3 = s0 * (w[:, 3:4] * (inv_v * inv_2))
    k48 = lax.broadcasted_iota(jnp.int32, (1, 48), 1) // 16
    s1rep = jnp.where(k48 == 0, s1c[0], jnp.where(k48 == 1, s1c[1], s1c[2]))
    a3rep = jnp.concatenate([a3, a3, a3], axis=1)
    v = a3rep * (s1rep * c2) + b4 * c3
    out_ref[:, MUL_S:D_IN] = _dot(v, q48[...])


def _fuse_weights(w1, w2, w3, w4):
    f32 = jnp.float32
    r_i = lax.broadcasted_iota(jnp.int32, (D_PAD, MUL_V), 0)
    i_i = lax.broadcasted_iota(jnp.int32, (D_PAD, MUL_V), 1)
    in_v = (r_i >= MUL_S) & (r_i < D_IN)
    ri = (r_i - MUL_S) // 3
    rc = (r_i - MUL_S) % 3
    g = (in_v & (ri == i_i)).astype(f32)
    r2g = g @ w2
    zpad = jnp.zeros((D_PAD - MUL_S, MUL_S), f32)
    m_w1 = jnp.concatenate([w1, zpad], axis=0)
    m_w3 = jnp.concatenate([w3, zpad[:, :MUL_V]], axis=0)
    m4 = [(in_v & (ri == i_i) & (rc == c)).astype(f32) @ w4 for c in range(3)]
    mfused = jnp.concatenate([m_w1] + m4 + [m_w3], axis=1)
    q_r = lax.broadcasted_iota(jnp.int32, (48, 48), 0)
    q_c = lax.broadcasted_iota(jnp.int32, (48, 48), 1)
    q48 = ((q_r // 16 == q_c % 3) & (q_r % 16 == q_c // 3)).astype(f32)
    return r2g, mfused, q48


def _tc_compute(x, edge_attr, edge_sh, wfc1, bfc1, wfc2, bfc2, r2g, mfused,
                q48):
    grid = x.shape[0] // _BE
    full = lambda s: pl.BlockSpec(s, lambda i: (0,) * len(s))
    return pl.pallas_call(
        _tc_body,
        grid=(grid,),
        in_specs=[
            pl.BlockSpec((_BE, D_PAD), lambda i: (i, 0)),
            pl.BlockSpec((_BE, D_EDGE), lambda i: (i, 0)),
            pl.BlockSpec((_BE, 4), lambda i: (i, 0)),
            full((D_EDGE, N_FEAT)), full((1, N_FEAT)),
            full((N_FEAT, 4)), full((1, 4)),
            full((D_PAD, MUL_S)), full((D_PAD, 96)), full((48, 48)),
        ],
        out_specs=pl.BlockSpec((_BE, D_PAD), lambda i: (i, 0)),
        out_shape=jax.ShapeDtypeStruct((x.shape[0], D_PAD), jnp.float32),
    )(x, edge_attr, edge_sh, wfc1, bfc1, wfc2, bfc2, r2g, mfused, q48)


_NT = 16
_EPT = N_EDGES // _NT
_KS = 128
_NFS = _EPT // _KS
_TS = _EPT - _NFS * _KS
_RPT = N_NODES // _NT
_WBR = 625
_PASSO = (0, 16, 24)
_WBO = (0, 0, 8)
_WBW = (16, 16, 8)
_EPC = N_EDGES // 2
_EPT3 = _EPC // _NT
_NF3 = _EPT3 // _KS
_T3 = _EPT3 - _NF3 * _KS


def _make_scatter(ne):
    ept = ne // _NT
    nfs = ept // _KS
    ts = ept - nfs * _KS
    assert nfs % 2 == 1 and ts % 8 == 0
    mesh = plsc.VectorSubcoreMesh(core_axis_name="c", subcore_axis_name="s")

    @functools.partial(
        pl.kernel,
        out_type=jax.ShapeDtypeStruct((2, N_NODES, HALF), jnp.float32),
        mesh=mesh,
        compiler_params=pltpu.CompilerParams(use_tc_tiling_on_sc=False),
        scratch_types=[
            pltpu.VMEM((_KS,), jnp.int32),
            pltpu.VMEM((_KS, 16), jnp.float32),
            pltpu.VMEM((_KS,), jnp.int32),
            pltpu.VMEM((_KS, 16), jnp.float32),
            pltpu.VMEM((ts,), jnp.int32),
            pltpu.VMEM((ts, 16), jnp.float32),
            pltpu.VMEM((_WBR, 16), jnp.float32),
            pltpu.VMEM((_WBR, 16), jnp.float32),
            pltpu.SemaphoreType.DMA,
            pltpu.SemaphoreType.DMA,
            pltpu.VMEM_SHARED((N_NODES, 16), jnp.float32),
        ],
    )
    def scatter_k(tp_hbm, src_hbm, sums_hbm,
                  idxa_v, dataa_v, idxb_v, datab_v, idxt_v, datat_v,
                  wb_v, z_v, sema, semb, acc):
        cid = lax.axis_index("c")
        sid = lax.axis_index("s")
        rbase = sid * _RPT
        base0 = sid * ept
        bufs = ((idxa_v, dataa_v, sema), (idxb_v, datab_v, semb))

        if True:
            def fill(i, carry):
                z_v[i] = jnp.zeros((16,), jnp.float32)
                return carry

            lax.fori_loop(0, _WBR, fill, 0)

            for p in range(3):
                for r in range(_RPT // _WBR):
                    pltpu.sync_copy(z_v, acc.at[pl.ds(rbase + r * _WBR, _WBR)])
                plsc.subcore_barrier()

                if True:
                    cstart = cid * HALF + _PASSO[p]

                    def load(j, idx_v, data_v):
                        base = pl.multiple_of(base0 + j * _KS, 8)
                        pltpu.sync_copy(src_hbm.at[pl.ds(base, _KS)], idx_v)
                        pltpu.sync_copy(
                            tp_hbm.at[pl.ds(base, _KS), pl.ds(cstart, 16)],
                            data_v)

                    load(0, idxa_v, dataa_v)
                    pltpu.async_copy(dataa_v, acc.at[idxa_v], sema, add=True)

                    def body(k, carry):
                        for b in range(2):
                            j = 2 * k + 1 + b
                            idx_v, data_v, sem = bufs[1 - b]
                            oidx, odata, osem = bufs[b]
                            load(j, idx_v, data_v)
                            pltpu.async_copy(data_v, acc.at[idx_v], sem,
                                             add=True)
                            pltpu.make_async_copy(
                                odata, acc.at[oidx], osem).wait()
                        return carry

                    lax.fori_loop(0, (nfs - 1) // 2, body, 0)
                    pltpu.make_async_copy(dataa_v, acc.at[idxa_v], sema).wait()

                    base = pl.multiple_of(base0 + nfs * _KS, 8)
                    pltpu.sync_copy(src_hbm.at[pl.ds(base, ts)], idxt_v)
                    pltpu.sync_copy(
                        tp_hbm.at[pl.ds(base, ts), pl.ds(cstart, 16)],
                        datat_v)
                    pltpu.sync_copy(datat_v, acc.at[idxt_v], add=True)
                plsc.subcore_barrier()

                for r in range(_RPT // _WBR):
                    rs = rbase + r * _WBR
                    pltpu.sync_copy(acc.at[pl.ds(rs, _WBR)], wb_v)
                    pltpu.sync_copy(
                        wb_v.at[:, pl.ds(_WBO[p], _WBW[p])],
                        sums_hbm.at[cid, pl.ds(rs, _WBR),
                                    pl.ds(_PASSO[p] + _WBO[p], _WBW[p])])

    return scatter_k


def _make_counts():
    mesh = plsc.VectorSubcoreMesh(core_axis_name="c", subcore_axis_name="s")

    @functools.partial(
        pl.kernel,
        out_type=jax.ShapeDtypeStruct((2, N_NODES, 8), jnp.float32),
        mesh=mesh,
        compiler_params=pltpu.CompilerParams(use_tc_tiling_on_sc=False),
        scratch_types=[
            pltpu.VMEM((_KS,), jnp.int32),
            pltpu.VMEM((_KS,), jnp.int32),
            pltpu.VMEM((_T3,), jnp.int32),
            pltpu.VMEM((_WBR, 16), jnp.float32),
            pltpu.VMEM((_KS, 16), jnp.float32),
            pltpu.SemaphoreType.DMA,
            pltpu.SemaphoreType.DMA,
            pltpu.VMEM_SHARED((N_NODES, 16), jnp.float32),
        ],
    )
    def counts_k(src_hbm, cnt_hbm,
                 idxa_v, idxb_v, idxt3_v, wb_v, ones_v, sema, semb, acc):
        cid = lax.axis_index("c")
        sid = lax.axis_index("s")
        rbase = sid * _RPT
        bufs = ((idxa_v, sema), (idxb_v, semb))

        def fill1(i, carry):
            ones_v[i] = jnp.ones((16,), jnp.float32)
            return carry

        lax.fori_loop(0, _KS, fill1, 0)

        def fillz(i, carry):
            wb_v[i] = jnp.zeros((16,), jnp.float32)
            return carry

        lax.fori_loop(0, _WBR, fillz, 0)
        for r in range(_RPT // _WBR):
            pltpu.sync_copy(wb_v, acc.at[pl.ds(rbase + r * _WBR, _WBR)])
        plsc.subcore_barrier()

        base3 = cid * _EPC + sid * _EPT3

        def start(j, idx_v, sem):
            base = pl.multiple_of(base3 + j * _KS, 8)
            pltpu.sync_copy(src_hbm.at[pl.ds(base, _KS)], idx_v)
            pltpu.async_copy(ones_v, acc.at[idx_v], sem, add=True)

        def finish(idx_v, sem):
            pltpu.make_async_copy(ones_v, acc.at[idx_v], sem).wait()

        start(0, *bufs[0])

        def body(k, carry):
            for b in range(2):
                start(2 * k + 1 + b, *bufs[1 - b])
                finish(*bufs[b])
            return carry

        lax.fori_loop(0, (_NF3 - 1) // 2, body, 0)
        finish(*bufs[0])
        base = pl.multiple_of(base3 + _NF3 * _KS, 8)
        pltpu.sync_copy(src_hbm.at[pl.ds(base, _T3)], idxt3_v)
        pltpu.sync_copy(ones_v.at[pl.ds(0, _T3)], acc.at[idxt3_v], add=True)
        plsc.subcore_barrier()

        for r in range(_RPT // _WBR):
            rs = rbase + r * _WBR
            pltpu.sync_copy(acc.at[pl.ds(rs, _WBR)], wb_v)
            pltpu.sync_copy(wb_v.at[:, pl.ds(0, 8)],
                            cnt_hbm.at[cid, pl.ds(rs, _WBR)])

    return counts_k


_BR = 2000


def _div_body(s1_ref, s2_ref, c_ref, o_ref):
    cnt = jnp.maximum(c_ref[0, :, 0:1] + c_ref[1, :, 0:1], 1.0)
    tot = jnp.concatenate([s1_ref[0] + s2_ref[0], s1_ref[1] + s2_ref[1]],
                          axis=1)
    o_ref[...] = tot / cnt


def _tc_divide(sums1, sums2, counts):
    grid = N_NODES // _BR
    return pl.pallas_call(
        _div_body,
        grid=(grid,),
        in_specs=[
            pl.BlockSpec((2, _BR, HALF), lambda i: (0, i, 0)),
            pl.BlockSpec((2, _BR, HALF), lambda i: (0, i, 0)),
            pl.BlockSpec((2, _BR, 8), lambda i: (0, i, 0)),
        ],
        out_specs=pl.BlockSpec((_BR, D_IN), lambda i: (i, 0)),
        out_shape=jax.ShapeDtypeStruct((N_NODES, D_IN), jnp.float32),
    )(sums1, sums2, counts)


def kernel(node_attr, edge_index, edge_attr, edge_sh,
           W_fc1, b_fc1, W_fc2, b_fc2, W1, W2, W3, W4):
    edge_src = edge_index[0].astype(jnp.int32)
    edge_dst = edge_index[1].astype(jnp.int32)
    node_pad = jnp.pad(node_attr, ((0, 0), (0, D_PAD - D_IN)))

    r2g, mfused, q48 = _fuse_weights(W1, W2, W3, W4)
    counts = _make_counts()(edge_src)

    sums = []
    e0 = 0
    for ne in _EH:
        dst_c = lax.slice_in_dim(edge_dst, e0, e0 + ne)
        src_c = lax.slice_in_dim(edge_src, e0, e0 + ne)
        ea_c = lax.slice_in_dim(edge_attr, e0, e0 + ne)
        sh_c = lax.slice_in_dim(edge_sh, e0, e0 + ne)
        x_c = _make_gather(ne)(node_pad, dst_c)
        tp_c = _tc_compute(x_c, ea_c, sh_c,
                           W_fc1, b_fc1.reshape(1, N_FEAT), W_fc2,
                           b_fc2.reshape(1, 4), r2g, mfused, q48)
        sums.append(_make_scatter(ne)(tp_c, src_c))
        e0 += ne
    return _tc_divide(sums[0], sums[1], counts)
